# baseline (speedup 1.0000x reference)
import functools

import jax
import jax.numpy as jnp
import numpy as np
from jax import lax
from jax.experimental import pallas as pl
from jax.experimental.pallas import tpu as pltpu
from jax.experimental.pallas import tpu_sc as plsc

N = 50000
E = 800000
DIM = 64
NBASIS = 16
NSPEC = 94
NLAYERS = 3
CUTOFF = 5.0
LOG2 = float(np.log(2.0))

NP2 = 51200
BN = 2048
BP = BN // 2
EB = 1536
EB2 = EB // 2
CPB = 8

NC = 2
NS = 16
HD = DIM // NC
CK = 192
G = 96
GR = CK // G
SCH = 8
NSUP = 33
NCHUNK = SCH * NSUP
EPAD = 16 * NCHUNK * CK
EPT = EPAD // NS
NPAD = 51200
RPT = NPAD // NS
ZR = 160


def _ssp(x):
    t = jnp.minimum(x * np.float32(1.4426950408889634), np.float32(126.0))
    return np.float32(LOG2) * (jnp.log2(1.0 + jnp.exp2(t)) - 1.0)


def _bd2(wm):
    z = jnp.zeros(wm.shape, jnp.float32)
    return jnp.concatenate(
        [jnp.concatenate([wm, z], axis=1),
         jnp.concatenate([z, wm], axis=1)], axis=0)


def _embed_body(sp_ref, wsp_ref, out_ref):
    sp = sp_ref[...]
    ii = lax.broadcasted_iota(jnp.int32, (BP, 128), 1)
    oh = jnp.concatenate(
        [(sp[:, 0][:, None] == ii).astype(jnp.float32),
         (sp[:, 1][:, None] == ii).astype(jnp.float32)], axis=1)
    out_ref[...] = jnp.dot(oh, _bd2(wsp_ref[...]),
                           preferred_element_type=jnp.float32)


def _embed(sp2, wsp_pad):
    return pl.pallas_call(
        _embed_body,
        grid=(NP2 // BN,),
        in_specs=[
            pl.BlockSpec((BP, 2), lambda i: (i, 0)),
            pl.BlockSpec((128, DIM), lambda i: (0, 0)),
        ],
        out_specs=pl.BlockSpec((BP, 128), lambda i: (i, 0)),
        out_shape=jax.ShapeDtypeStruct((NP2 // 2, 128), jnp.float32),
    )(sp2, wsp_pad)


def _filter_body(d_ref, sw_ref, f0w_ref, f0b_ref, f1w_ref, f1b_ref,
                 f2w_ref, f2b_ref, out_ref):
    d = d_ref[0, 0, :]
    sw = sw_ref[0, 0, :]
    de, do = d[:EB2], d[EB2:]
    swe, swo = sw[:EB2], sw[EB2:]
    delta = CUTOFF / NBASIS
    gamma = 1.0 / (2.0 * delta * delta)
    mus = lax.broadcasted_iota(jnp.int32, (EB2, NBASIS), 1).astype(jnp.float32) \
        * (CUTOFF / (NBASIS - 1))
    rb2 = jnp.concatenate(
        [jnp.exp(-gamma * (de[:, None] - mus) ** 2),
         jnp.exp(-gamma * (do[:, None] - mus) ** 2)], axis=1)
    swb = jnp.concatenate(
        [jnp.broadcast_to(swe[:, None], (EB2, DIM)),
         jnp.broadcast_to(swo[:, None], (EB2, DIM))], axis=1)

    def bd(wm, rows, cols):
        z = jnp.zeros((rows, cols), jnp.float32)
        return jnp.concatenate(
            [jnp.concatenate([wm, z], axis=1),
             jnp.concatenate([z, wm], axis=1)], axis=0)

    b0 = jnp.concatenate([f0b_ref[0, :], f0b_ref[0, :]])[None, :]
    b1 = jnp.concatenate([f1b_ref[0, :], f1b_ref[0, :]])[None, :]
    b2 = jnp.concatenate([f2b_ref[0, :], f2b_ref[0, :]])[None, :]
    h = _ssp(jnp.dot(rb2, bd(f0w_ref[...], NBASIS, 64),
                     preferred_element_type=jnp.float32) + b0)
    h = _ssp(jnp.dot(h, bd(f1w_ref[...], 64, 64),
                     preferred_element_type=jnp.float32) + b1)
    w = jnp.dot(h, bd(f2w_ref[...], 64, DIM),
                preferred_element_type=jnp.float32) + b2
    out_ref[...] = _ssp(w) * swb


def _filter_layer(d3, s3, f0w, f0b, f1w, f1b, f2w, f2b):
    wspec = lambda s: pl.BlockSpec(s, lambda i: tuple(0 for _ in s))
    espec = pl.BlockSpec((1, 1, EB), lambda i: (i, 0, 0))
    return pl.pallas_call(
        _filter_body,
        grid=(EPAD // EB,),
        in_specs=[
            espec, espec,
            wspec((NBASIS, 64)), wspec((1, 64)),
            wspec((64, 64)), wspec((1, 64)),
            wspec((64, DIM)), wspec((1, DIM)),
        ],
        out_specs=pl.BlockSpec((EB2, 128), lambda i: (i, 0)),
        out_shape=jax.ShapeDtypeStruct((EPAD // 2, 128), jnp.float32),
    )(d3, s3, f0w, f0b.reshape(1, 64), f1w, f1b.reshape(1, 64),
      f2w, f2b.reshape(1, DIM))


def _node_in_body(x_ref, w_ref, b_ref, out_ref):
    bt = jnp.concatenate([b_ref[0, :], b_ref[0, :]])[None, :]
    out_ref[...] = jnp.dot(x_ref[...], _bd2(w_ref[...]),
                           preferred_element_type=jnp.float32) + bt


def _node_in(x2p, w, b):
    return pl.pallas_call(
        _node_in_body,
        grid=(NP2 // BN,),
        in_specs=[
            pl.BlockSpec((BP, 128), lambda i: (i, 0)),
            pl.BlockSpec((DIM, DIM), lambda i: (0, 0)),
            pl.BlockSpec((1, DIM), lambda i: (0, 0)),
        ],
        out_specs=pl.BlockSpec((BP, 128), lambda i: (i, 0)),
        out_shape=jax.ShapeDtypeStruct((NP2 // 2, 128), jnp.float32),
    )(x2p, w, b.reshape(1, DIM))


def _node_out_body(acc_ref, xp_ref, w2_ref, b2_ref, w3_ref, b3_ref, out_ref):
    b2t = jnp.concatenate([b2_ref[0, :], b2_ref[0, :]])[None, :]
    b3t = jnp.concatenate([b3_ref[0, :], b3_ref[0, :]])[None, :]
    u = _ssp(jnp.dot(acc_ref[...], _bd2(w2_ref[...]),
                     preferred_element_type=jnp.float32) + b2t)
    out_ref[...] = jnp.dot(u, _bd2(w3_ref[...]),
                           preferred_element_type=jnp.float32) + b3t + xp_ref[...]


def _node_out(acc2p, xp2p, w2, b2, w3, b3):
    return pl.pallas_call(
        _node_out_body,
        grid=(NP2 // BN,),
        in_specs=[
            pl.BlockSpec((BP, 128), lambda i: (i, 0)),
            pl.BlockSpec((BP, 128), lambda i: (i, 0)),
            pl.BlockSpec((DIM, DIM), lambda i: (0, 0)),
            pl.BlockSpec((1, DIM), lambda i: (0, 0)),
            pl.BlockSpec((DIM, DIM), lambda i: (0, 0)),
            pl.BlockSpec((1, DIM), lambda i: (0, 0)),
        ],
        out_specs=pl.BlockSpec((BP, 128), lambda i: (i, 0)),
        out_shape=jax.ShapeDtypeStruct((NP2 // 2, 128), jnp.float32),
    )(acc2p, xp2p, w2, b2.reshape(1, DIM), w3, b3.reshape(1, DIM))


def _sc_conv_body(xi2_hbm, wp_hbm, srcr_hbm, dstr_hbm, out_hbm,
                  sidx, didx, g0, g1, w0, w1, acc,
                  sem0, sem1, ssem0, ssem1):
    c = lax.axis_index("c")
    s = lax.axis_index("s")
    gb = (g0, g1)
    wb = (w0, w1)
    sems = (sem0, sem1)
    ssems = (ssem0, ssem1)

    zv = jnp.zeros((16,), jnp.float32)

    def zb_body(r, zc):
        g0[r, pl.ds(0, 16)] = zv
        g0[r, pl.ds(16, 16)] = zv
        return zc
    lax.fori_loop(0, ZR, zb_body, 0, unroll=8)
    for q in range(RPT // ZR):
        z0 = pl.multiple_of(s * RPT + q * ZR, ZR)
        pltpu.sync_copy(g0.at[pl.ds(0, ZR)], acc.at[pl.ds(z0, ZR)])
    plsc.subcore_barrier()

    tbase = s * EPT

    def load_idx(k):
        rb = pl.multiple_of((tbase + k * SCH * CK) // G, GR * SCH)
        pltpu.sync_copy(srcr_hbm.at[pl.ds(rb, GR * SCH)], sidx)
        pltpu.sync_copy(dstr_hbm.at[c, pl.ds(rb, GR * SCH)], didx)

    def wp_slice(k, p):
        ci = s * NCHUNK + k * SCH + p
        b = lax.shift_right_logical(ci, 3)
        cb = lax.bitwise_and(ci, 7)
        q = lax.shift_right_logical(cb, 2)
        pstart = pl.multiple_of(b * EB2 + lax.bitwise_and(cb, 3) * CK, CK)
        return wp_hbm.at[pl.ds(pstart, CK), q, pl.ds(c * HD, HD)]

    def drain_scatter(par):
        for j in range(GR):
            pltpu.make_async_copy(gb[par].at[pl.ds(j * G, G)],
                                  acc.at[sidx.at[j]], ssems[par]).wait()

    def issue(k, p, par, drain):
        if drain:
            drain_scatter(par)
        for j in range(GR):
            pltpu.async_copy(xi2_hbm.at[didx.at[GR * p + j]],
                             gb[par].at[pl.ds(j * G, G)], sems[par])
        pltpu.async_copy(wp_slice(k, p), wb[par], sems[par])

    def wait(k, p, par):
        for j in range(GR):
            pltpu.make_async_copy(xi2_hbm.at[didx.at[GR * p + j]],
                                  gb[par].at[pl.ds(j * G, G)], sems[par]).wait()
        pltpu.make_async_copy(wp_slice(k, p), wb[par], sems[par]).wait()

    load_idx(0)
    issue(0, 0, 0, False)

    def sup(k, carry):
        for p in range(SCH):
            par = p & 1
            if p < SCH - 1:
                issue(k, p + 1, 1 - par, p != 0)
            wait(k, p, par)
            g, w = gb[par], wb[par]

            def mul(r0, mc):
                for u in range(8):
                    r = r0 * 8 + u
                    g[r, pl.ds(0, 16)] = g[r, pl.ds(0, 16)] * w[r, pl.ds(0, 16)]
                    g[r, pl.ds(16, 16)] = g[r, pl.ds(16, 16)] * w[r, pl.ds(16, 16)]
                return mc
            lax.fori_loop(0, CK // 8, mul, 0)

            for j in range(GR):
                pltpu.async_copy(g.at[pl.ds(j * G, G)],
                                 acc.at[sidx.at[GR * p + j]], ssems[par],
                                 add=True)
            if p == SCH - 1:
                @pl.when(k + 1 < NSUP)
                def _next():
                    drain_scatter(par)
                    load_idx(k + 1)
                    issue(k + 1, 0, 1 - par, True)
        return carry

    lax.fori_loop(0, NSUP, sup, 0)
    drain_scatter(0)
    drain_scatter(1)
    plsc.subcore_barrier()
    for q in range(RPT // ZR):
        r0 = pl.multiple_of(s * RPT + q * ZR, ZR)
        pltpu.sync_copy(acc.at[pl.ds(r0, ZR)], out_hbm.at[pl.ds(r0, ZR), c])


def _sc_conv(xi_h, wp, srcr, dstr):
    mesh = plsc.VectorSubcoreMesh(core_axis_name="c", subcore_axis_name="s")
    return pl.kernel(
        _sc_conv_body,
        out_type=jax.ShapeDtypeStruct((NPAD, NC, HD), jnp.float32),
        mesh=mesh,
        compiler_params=pltpu.CompilerParams(use_tc_tiling_on_sc=False),
        scratch_types=[
            pltpu.VMEM((GR * SCH, G), jnp.int32),
            pltpu.VMEM((GR * SCH, G), jnp.int32),
            pltpu.VMEM((CK, HD), jnp.float32),
            pltpu.VMEM((CK, HD), jnp.float32),
            pltpu.VMEM((CK, HD), jnp.float32),
            pltpu.VMEM((CK, HD), jnp.float32),
            pltpu.VMEM_SHARED((NPAD, HD), jnp.float32),
            pltpu.SemaphoreType.DMA,
            pltpu.SemaphoreType.DMA,
            pltpu.SemaphoreType.DMA,
            pltpu.SemaphoreType.DMA,
        ],
    )(xi_h.reshape(2 * NP2, HD), wp.reshape(EPAD // 2, 2, DIM), srcr, dstr)


def kernel(species, edge_src, edge_dst, distances, switch,
           W_sp, aw1_W, aw1_b, f0_W, f0_b, f1_W, f1_b, f2_W, f2_b,
           aw2_W, aw2_b, aw3_W, aw3_b):
    species = species.astype(jnp.int32)
    edge_src = edge_src.astype(jnp.int32)
    edge_dst = edge_dst.astype(jnp.int32)

    wsp_pad = jnp.zeros((128, DIM), jnp.float32).at[:NSPEC].set(W_sp)
    sp2 = jnp.pad(species, (0, NP2 - N)).reshape(NP2 // 2, 2)
    xi2p = _embed(sp2, wsp_pad)

    d3 = jnp.pad(distances, (0, EPAD - E)).reshape(EPAD // EB, 1, EB)
    s3 = jnp.pad(switch, (0, EPAD - E)).reshape(EPAD // EB, 1, EB)
    wps = [_filter_layer(d3, s3, f0_W[l], f0_b[l], f1_W[l], f1_b[l],
                         f2_W[l], f2_b[l]) for l in range(NLAYERS)]

    srcr = jnp.pad(edge_src, (0, EPAD - E)).reshape(EPAD // G, G)
    dstr = jnp.pad(jnp.stack([2 * edge_dst, 2 * edge_dst + 1]),
                   ((0, 0), (0, EPAD - E))).reshape(NC, EPAD // G, G)

    for l in range(NLAYERS):
        y2p = _node_in(xi2p, aw1_W[l], aw1_b[l])
        acc = _sc_conv(y2p, wps[l], srcr, dstr)
        xi2p = _node_out(acc.reshape(NP2 // 2, 128), xi2p,
                         aw2_W[l], aw2_b[l], aw3_W[l], aw3_b[l])
    return xi2p.reshape(NP2, DIM)[:N]


# speedup vs baseline: 2.1008x; 2.1008x over previous
import functools

import jax
import jax.numpy as jnp
import numpy as np
from jax import lax
from jax.experimental import pallas as pl
from jax.experimental.pallas import tpu as pltpu
from jax.experimental.pallas import tpu_sc as plsc

N = 50000
E = 800000
DIM = 64
NBASIS = 16
NSPEC = 94
NLAYERS = 3
CUTOFF = 5.0
LOG2 = float(np.log(2.0))

NP2 = 51200
BN = 2048
BP = BN // 2
EB = 1536
EB2 = EB // 2
CPB = 8

NC = 2
NS = 16
HD = DIM // NC
CK = 192
G = 96
GR = CK // G
SCH = 8
NSUP = 33
NCHUNK = SCH * NSUP
EPAD = 16 * NCHUNK * CK
EPT = EPAD // NS
NPAD = 51200
RPT = NPAD // NS
ZR = 160


def _ssp(x):
    t = jnp.minimum(x * np.float32(1.4426950408889634), np.float32(126.0))
    return np.float32(LOG2) * (jnp.log2(1.0 + jnp.exp2(t)) - 1.0)


def _bd2(wm):
    z = jnp.zeros(wm.shape, jnp.float32)
    return jnp.concatenate(
        [jnp.concatenate([wm, z], axis=1),
         jnp.concatenate([z, wm], axis=1)], axis=0)


def _embed_body(sp_ref, wsp_ref, out_ref):
    sp = sp_ref[...]
    ii = lax.broadcasted_iota(jnp.int32, (BP, 128), 1)
    oh = jnp.concatenate(
        [(sp[:, 0][:, None] == ii).astype(jnp.float32),
         (sp[:, 1][:, None] == ii).astype(jnp.float32)], axis=1)
    out_ref[...] = jnp.dot(oh, _bd2(wsp_ref[...]),
                           preferred_element_type=jnp.float32)


def _embed(sp2, wsp_pad):
    return pl.pallas_call(
        _embed_body,
        grid=(NP2 // BN,),
        in_specs=[
            pl.BlockSpec((BP, 2), lambda i: (i, 0)),
            pl.BlockSpec((128, DIM), lambda i: (0, 0)),
        ],
        out_specs=pl.BlockSpec((BP, 128), lambda i: (i, 0)),
        out_shape=jax.ShapeDtypeStruct((NP2 // 2, 128), jnp.float32),
    )(sp2, wsp_pad)


def _filter_body(d_ref, sw_ref, f0w_ref, f0b_ref, f1w_ref, f1b_ref,
                 f2w_ref, f2b_ref, *out_refs):
    d = d_ref[0, 0, :]
    sw = sw_ref[0, 0, :]
    de, do = d[:EB2], d[EB2:]
    swe, swo = sw[:EB2], sw[EB2:]
    delta = CUTOFF / NBASIS
    gamma = 1.0 / (2.0 * delta * delta)
    mus = lax.broadcasted_iota(jnp.int32, (EB2, NBASIS), 1).astype(jnp.float32) \
        * (CUTOFF / (NBASIS - 1))
    rb2 = jnp.concatenate(
        [jnp.exp(-gamma * (de[:, None] - mus) ** 2),
         jnp.exp(-gamma * (do[:, None] - mus) ** 2)], axis=1)
    swb = jnp.concatenate(
        [jnp.broadcast_to(swe[:, None], (EB2, DIM)),
         jnp.broadcast_to(swo[:, None], (EB2, DIM))], axis=1)

    def bd(wm, rows, cols):
        z = jnp.zeros((rows, cols), jnp.float32)
        return jnp.concatenate(
            [jnp.concatenate([wm, z], axis=1),
             jnp.concatenate([z, wm], axis=1)], axis=0)

    for l in range(NLAYERS):
        b0 = jnp.concatenate([f0b_ref[l], f0b_ref[l]])[None, :]
        b1 = jnp.concatenate([f1b_ref[l], f1b_ref[l]])[None, :]
        b2 = jnp.concatenate([f2b_ref[l], f2b_ref[l]])[None, :]
        h = _ssp(jnp.dot(rb2, bd(f0w_ref[l], NBASIS, 64),
                         preferred_element_type=jnp.float32) + b0)
        h = _ssp(jnp.dot(h, bd(f1w_ref[l], 64, 64),
                         preferred_element_type=jnp.float32) + b1)
        w = jnp.dot(h, bd(f2w_ref[l], 64, DIM),
                    preferred_element_type=jnp.float32) + b2
        out_refs[l][...] = _ssp(w) * swb


def _filter(d3, s3, f0_W, f0_b, f1_W, f1_b, f2_W, f2_b):
    wspec = lambda s: pl.BlockSpec(s, lambda i: tuple(0 for _ in s))
    espec = pl.BlockSpec((1, 1, EB), lambda i: (i, 0, 0))
    oshape = jax.ShapeDtypeStruct((EPAD // 2, 128), jnp.float32)
    return pl.pallas_call(
        _filter_body,
        grid=(EPAD // EB,),
        in_specs=[
            espec, espec,
            wspec((NLAYERS, NBASIS, 64)), wspec((NLAYERS, 64)),
            wspec((NLAYERS, 64, 64)), wspec((NLAYERS, 64)),
            wspec((NLAYERS, 64, DIM)), wspec((NLAYERS, DIM)),
        ],
        out_specs=[pl.BlockSpec((EB2, 128), lambda i: (i, 0))] * NLAYERS,
        out_shape=[oshape] * NLAYERS,
    )(d3, s3, f0_W, f0_b, f1_W, f1_b, f2_W, f2_b)


def _node_in_body(x_ref, w_ref, b_ref, out_ref):
    bt = jnp.concatenate([b_ref[0, :], b_ref[0, :]])[None, :]
    out_ref[...] = jnp.dot(x_ref[...], _bd2(w_ref[...]),
                           preferred_element_type=jnp.float32) + bt


def _node_in(x2p, w, b):
    return pl.pallas_call(
        _node_in_body,
        grid=(NP2 // BN,),
        in_specs=[
            pl.BlockSpec((BP, 128), lambda i: (i, 0)),
            pl.BlockSpec((DIM, DIM), lambda i: (0, 0)),
            pl.BlockSpec((1, DIM), lambda i: (0, 0)),
        ],
        out_specs=pl.BlockSpec((BP, 128), lambda i: (i, 0)),
        out_shape=jax.ShapeDtypeStruct((NP2 // 2, 128), jnp.float32),
    )(x2p, w, b.reshape(1, DIM))


def _node_out_body(acc_ref, xp_ref, w2_ref, b2_ref, w3_ref, b3_ref, out_ref):
    b2t = jnp.concatenate([b2_ref[0, :], b2_ref[0, :]])[None, :]
    b3t = jnp.concatenate([b3_ref[0, :], b3_ref[0, :]])[None, :]
    u = _ssp(jnp.dot(acc_ref[...], _bd2(w2_ref[...]),
                     preferred_element_type=jnp.float32) + b2t)
    out_ref[...] = jnp.dot(u, _bd2(w3_ref[...]),
                           preferred_element_type=jnp.float32) + b3t + xp_ref[...]


def _node_out(acc2p, xp2p, w2, b2, w3, b3):
    return pl.pallas_call(
        _node_out_body,
        grid=(NP2 // BN,),
        in_specs=[
            pl.BlockSpec((BP, 128), lambda i: (i, 0)),
            pl.BlockSpec((BP, 128), lambda i: (i, 0)),
            pl.BlockSpec((DIM, DIM), lambda i: (0, 0)),
            pl.BlockSpec((1, DIM), lambda i: (0, 0)),
            pl.BlockSpec((DIM, DIM), lambda i: (0, 0)),
            pl.BlockSpec((1, DIM), lambda i: (0, 0)),
        ],
        out_specs=pl.BlockSpec((BP, 128), lambda i: (i, 0)),
        out_shape=jax.ShapeDtypeStruct((NP2 // 2, 128), jnp.float32),
    )(acc2p, xp2p, w2, b2.reshape(1, DIM), w3, b3.reshape(1, DIM))


def _sc_conv_body(xi2_hbm, wp_hbm, srcr_hbm, dstr_hbm, out_hbm,
                  sidx, didx, g0, g1, w0, w1, acc,
                  sem0, sem1, ssem0, ssem1):
    c = lax.axis_index("c")
    s = lax.axis_index("s")
    gb = (g0, g1)
    wb = (w0, w1)
    sems = (sem0, sem1)
    ssems = (ssem0, ssem1)

    zv = jnp.zeros((16,), jnp.float32)

    def zb_body(r, zc):
        g0[r, pl.ds(0, 16)] = zv
        g0[r, pl.ds(16, 16)] = zv
        return zc
    lax.fori_loop(0, ZR, zb_body, 0, unroll=8)
    for q in range(RPT // ZR):
        z0 = pl.multiple_of(s * RPT + q * ZR, ZR)
        pltpu.sync_copy(g0.at[pl.ds(0, ZR)], acc.at[pl.ds(z0, ZR)])
    plsc.subcore_barrier()

    tbase = s * EPT

    def load_idx(k):
        rb = pl.multiple_of((tbase + k * SCH * CK) // G, GR * SCH)
        pltpu.sync_copy(srcr_hbm.at[pl.ds(rb, GR * SCH)], sidx)
        pltpu.sync_copy(dstr_hbm.at[c, pl.ds(rb, GR * SCH)], didx)

    def wp_slice(k, p):
        ci = s * NCHUNK + k * SCH + p
        b = lax.shift_right_logical(ci, 3)
        cb = lax.bitwise_and(ci, 7)
        q = lax.shift_right_logical(cb, 2)
        pstart = pl.multiple_of(b * EB2 + lax.bitwise_and(cb, 3) * CK, CK)
        return wp_hbm.at[pl.ds(pstart, CK), pl.ds(q * 64 + c * HD, HD)]

    def drain_scatter(par):
        for j in range(GR):
            pltpu.make_async_copy(gb[par].at[pl.ds(j * G, G)],
                                  acc.at[sidx.at[j]], ssems[par]).wait()

    def issue(k, p, par, drain):
        if drain:
            drain_scatter(par)
        for j in range(GR):
            pltpu.async_copy(xi2_hbm.at[didx.at[GR * p + j]],
                             gb[par].at[pl.ds(j * G, G)], sems[par])
        pltpu.async_copy(wp_slice(k, p), wb[par], sems[par])

    def wait(k, p, par):
        for j in range(GR):
            pltpu.make_async_copy(xi2_hbm.at[didx.at[GR * p + j]],
                                  gb[par].at[pl.ds(j * G, G)], sems[par]).wait()
        pltpu.make_async_copy(wp_slice(k, p), wb[par], sems[par]).wait()

    load_idx(0)
    issue(0, 0, 0, False)

    def sup(k, carry):
        for p in range(SCH):
            par = p & 1
            if p < SCH - 1:
                issue(k, p + 1, 1 - par, p != 0)
            wait(k, p, par)
            g, w = gb[par], wb[par]

            def mul(r0, mc):
                for u in range(8):
                    r = r0 * 8 + u
                    g[r, pl.ds(0, 16)] = g[r, pl.ds(0, 16)] * w[r, pl.ds(0, 16)]
                    g[r, pl.ds(16, 16)] = g[r, pl.ds(16, 16)] * w[r, pl.ds(16, 16)]
                return mc
            lax.fori_loop(0, CK // 8, mul, 0)

            for j in range(GR):
                pltpu.async_copy(g.at[pl.ds(j * G, G)],
                                 acc.at[sidx.at[GR * p + j]], ssems[par],
                                 add=True)
            if p == SCH - 1:
                @pl.when(k + 1 < NSUP)
                def _next():
                    drain_scatter(par)
                    load_idx(k + 1)
                    issue(k + 1, 0, 1 - par, True)
        return carry

    lax.fori_loop(0, NSUP, sup, 0)
    drain_scatter(0)
    drain_scatter(1)
    plsc.subcore_barrier()
    for q in range(RPT // ZR):
        r0 = pl.multiple_of(s * RPT + q * ZR, ZR)
        pltpu.sync_copy(acc.at[pl.ds(r0, ZR)], out_hbm.at[pl.ds(r0, ZR), c])


def _sc_conv(xi_h, wp, srcr, dstr):
    mesh = plsc.VectorSubcoreMesh(core_axis_name="c", subcore_axis_name="s")
    return pl.kernel(
        _sc_conv_body,
        out_type=jax.ShapeDtypeStruct((NPAD, NC, HD), jnp.float32),
        mesh=mesh,
        compiler_params=pltpu.CompilerParams(use_tc_tiling_on_sc=False),
        scratch_types=[
            pltpu.VMEM((GR * SCH, G), jnp.int32),
            pltpu.VMEM((GR * SCH, G), jnp.int32),
            pltpu.VMEM((CK, HD), jnp.float32),
            pltpu.VMEM((CK, HD), jnp.float32),
            pltpu.VMEM((CK, HD), jnp.float32),
            pltpu.VMEM((CK, HD), jnp.float32),
            pltpu.VMEM_SHARED((NPAD, HD), jnp.float32),
            pltpu.SemaphoreType.DMA,
            pltpu.SemaphoreType.DMA,
            pltpu.SemaphoreType.DMA,
            pltpu.SemaphoreType.DMA,
        ],
    )(xi_h.reshape(2 * NP2, HD), wp, srcr, dstr)


def kernel(species, edge_src, edge_dst, distances, switch,
           W_sp, aw1_W, aw1_b, f0_W, f0_b, f1_W, f1_b, f2_W, f2_b,
           aw2_W, aw2_b, aw3_W, aw3_b):
    species = species.astype(jnp.int32)
    edge_src = edge_src.astype(jnp.int32)
    edge_dst = edge_dst.astype(jnp.int32)

    wsp_pad = jnp.zeros((128, DIM), jnp.float32).at[:NSPEC].set(W_sp)
    sp2 = jnp.pad(species, (0, NP2 - N)).reshape(NP2 // 2, 2)
    xi2p = _embed(sp2, wsp_pad)

    d3 = jnp.pad(distances, (0, EPAD - E)).reshape(EPAD // EB, 1, EB)
    s3 = jnp.pad(switch, (0, EPAD - E)).reshape(EPAD // EB, 1, EB)
    wps = _filter(d3, s3, f0_W, f0_b, f1_W, f1_b, f2_W, f2_b)

    srcr = jnp.pad(edge_src, (0, EPAD - E)).reshape(EPAD // G, G)
    dstr = jnp.pad(jnp.stack([2 * edge_dst, 2 * edge_dst + 1]),
                   ((0, 0), (0, EPAD - E))).reshape(NC, EPAD // G, G)

    for l in range(NLAYERS):
        y2p = _node_in(xi2p, aw1_W[l], aw1_b[l])
        acc = _sc_conv(y2p, wps[l], srcr, dstr)
        xi2p = _node_out(acc.reshape(NP2 // 2, 128), xi2p,
                         aw2_W[l], aw2_b[l], aw3_W[l], aw3_b[l])
    return xi2p.reshape(NP2, DIM)[:N]


# speedup vs baseline: 2.3065x; 1.0979x over previous
import functools

import jax
import jax.numpy as jnp
import numpy as np
from jax import lax
from jax.experimental import pallas as pl
from jax.experimental.pallas import tpu as pltpu
from jax.experimental.pallas import tpu_sc as plsc

N = 50000
E = 800000
DIM = 64
NBASIS = 16
NSPEC = 94
NLAYERS = 3
CUTOFF = 5.0
LOG2 = float(np.log(2.0))

NP2 = 51200
BN = 2048
BP = BN // 2
EB = 1536
EB2 = EB // 2
CPB = 8

NC = 2
NS = 16
HD = DIM // NC
CK = 192
G = 96
GR = CK // G
SCH = 8
NSUP = 33
NCHUNK = SCH * NSUP
EPAD = 16 * NCHUNK * CK
EPT = EPAD // NS
NPAD = 51200
RPT = NPAD // NS
ZR = 160


def _ssp(x):
    t = jnp.minimum(x * np.float32(1.4426950408889634), np.float32(126.0))
    return np.float32(LOG2) * (jnp.log2(1.0 + jnp.exp2(t)) - 1.0)


def _bd2(wm):
    z = jnp.zeros(wm.shape, jnp.float32)
    return jnp.concatenate(
        [jnp.concatenate([wm, z], axis=1),
         jnp.concatenate([z, wm], axis=1)], axis=0)


def _embed_body(sp_ref, wsp_ref, out_ref):
    sp = sp_ref[...]
    ii = lax.broadcasted_iota(jnp.int32, (BP, 128), 1)
    oh = jnp.concatenate(
        [(sp[:, 0][:, None] == ii).astype(jnp.float32),
         (sp[:, 1][:, None] == ii).astype(jnp.float32)], axis=1)
    out_ref[...] = jnp.dot(oh, _bd2(wsp_ref[...]),
                           preferred_element_type=jnp.float32)


def _embed(sp2, wsp_pad):
    return pl.pallas_call(
        _embed_body,
        grid=(NP2 // BN,),
        in_specs=[
            pl.BlockSpec((BP, 2), lambda i: (i, 0)),
            pl.BlockSpec((128, DIM), lambda i: (0, 0)),
        ],
        out_specs=pl.BlockSpec((BP, 128), lambda i: (i, 0)),
        out_shape=jax.ShapeDtypeStruct((NP2 // 2, 128), jnp.float32),
    )(sp2, wsp_pad)


def _filter_body(LAYERS, d_ref, sw_ref, f0w_ref, f0b_ref, f1w_ref, f1b_ref,
                 f2w_ref, f2b_ref, *out_refs):
    d = d_ref[0, 0, :]
    sw = sw_ref[0, 0, :]
    de, do = d[:EB2], d[EB2:]
    swe, swo = sw[:EB2], sw[EB2:]
    delta = CUTOFF / NBASIS
    gamma = 1.0 / (2.0 * delta * delta)
    mus = lax.broadcasted_iota(jnp.int32, (EB2, NBASIS), 1).astype(jnp.float32) \
        * (CUTOFF / (NBASIS - 1))
    rb2 = jnp.concatenate(
        [jnp.exp(-gamma * (de[:, None] - mus) ** 2),
         jnp.exp(-gamma * (do[:, None] - mus) ** 2)], axis=1)
    swb = jnp.concatenate(
        [jnp.broadcast_to(swe[:, None], (EB2, DIM)),
         jnp.broadcast_to(swo[:, None], (EB2, DIM))], axis=1)

    def bd(wm, rows, cols):
        z = jnp.zeros((rows, cols), jnp.float32)
        return jnp.concatenate(
            [jnp.concatenate([wm, z], axis=1),
             jnp.concatenate([z, wm], axis=1)], axis=0)

    for o, l in enumerate(LAYERS):
        b0 = jnp.concatenate([f0b_ref[l], f0b_ref[l]])[None, :]
        b1 = jnp.concatenate([f1b_ref[l], f1b_ref[l]])[None, :]
        b2 = jnp.concatenate([f2b_ref[l], f2b_ref[l]])[None, :]
        h = _ssp(jnp.dot(rb2, bd(f0w_ref[l], NBASIS, 64),
                         preferred_element_type=jnp.float32) + b0)
        h = _ssp(jnp.dot(h, bd(f1w_ref[l], 64, 64),
                         preferred_element_type=jnp.float32) + b1)
        w = jnp.dot(h, bd(f2w_ref[l], 64, DIM),
                    preferred_element_type=jnp.float32) + b2
        out_refs[o][...] = _ssp(w) * swb


def _filter(layers, d3, s3, f0_W, f0_b, f1_W, f1_b, f2_W, f2_b):
    wspec = lambda s: pl.BlockSpec(s, lambda i: tuple(0 for _ in s))
    espec = pl.BlockSpec((1, 1, EB), lambda i: (i, 0, 0))
    oshape = jax.ShapeDtypeStruct((EPAD // 2, 128), jnp.float32)
    return pl.pallas_call(
        functools.partial(_filter_body, tuple(layers)),
        grid=(EPAD // EB,),
        in_specs=[
            espec, espec,
            wspec((NLAYERS, NBASIS, 64)), wspec((NLAYERS, 64)),
            wspec((NLAYERS, 64, 64)), wspec((NLAYERS, 64)),
            wspec((NLAYERS, 64, DIM)), wspec((NLAYERS, DIM)),
        ],
        out_specs=[pl.BlockSpec((EB2, 128), lambda i: (i, 0))] * len(layers),
        out_shape=[oshape] * len(layers),
    )(d3, s3, f0_W, f0_b, f1_W, f1_b, f2_W, f2_b)


def _node_in_body(x_ref, w_ref, b_ref, out_ref):
    bt = jnp.concatenate([b_ref[0, :], b_ref[0, :]])[None, :]
    out_ref[...] = jnp.dot(x_ref[...], _bd2(w_ref[...]),
                           preferred_element_type=jnp.float32) + bt


def _node_in(x2p, w, b):
    return pl.pallas_call(
        _node_in_body,
        grid=(NP2 // BN,),
        in_specs=[
            pl.BlockSpec((BP, 128), lambda i: (i, 0)),
            pl.BlockSpec((DIM, DIM), lambda i: (0, 0)),
            pl.BlockSpec((1, DIM), lambda i: (0, 0)),
        ],
        out_specs=pl.BlockSpec((BP, 128), lambda i: (i, 0)),
        out_shape=jax.ShapeDtypeStruct((NP2 // 2, 128), jnp.float32),
    )(x2p, w, b.reshape(1, DIM))


def _node_out_body(acc_ref, xp_ref, w2_ref, b2_ref, w3_ref, b3_ref, out_ref):
    b2t = jnp.concatenate([b2_ref[0, :], b2_ref[0, :]])[None, :]
    b3t = jnp.concatenate([b3_ref[0, :], b3_ref[0, :]])[None, :]
    u = _ssp(jnp.dot(acc_ref[...], _bd2(w2_ref[...]),
                     preferred_element_type=jnp.float32) + b2t)
    out_ref[...] = jnp.dot(u, _bd2(w3_ref[...]),
                           preferred_element_type=jnp.float32) + b3t + xp_ref[...]


def _node_out(acc2p, xp2p, w2, b2, w3, b3):
    return pl.pallas_call(
        _node_out_body,
        grid=(NP2 // BN,),
        in_specs=[
            pl.BlockSpec((BP, 128), lambda i: (i, 0)),
            pl.BlockSpec((BP, 128), lambda i: (i, 0)),
            pl.BlockSpec((DIM, DIM), lambda i: (0, 0)),
            pl.BlockSpec((1, DIM), lambda i: (0, 0)),
            pl.BlockSpec((DIM, DIM), lambda i: (0, 0)),
            pl.BlockSpec((1, DIM), lambda i: (0, 0)),
        ],
        out_specs=pl.BlockSpec((BP, 128), lambda i: (i, 0)),
        out_shape=jax.ShapeDtypeStruct((NP2 // 2, 128), jnp.float32),
    )(acc2p, xp2p, w2, b2.reshape(1, DIM), w3, b3.reshape(1, DIM))


def _sc_conv_body(xi2_hbm, wp_hbm, srcr_hbm, dstr_hbm, out_hbm,
                  sidx, didx, g0, g1, w0, w1, acc,
                  sem0, sem1, ssem0, ssem1):
    c = lax.axis_index("c")
    s = lax.axis_index("s")
    gb = (g0, g1)
    wb = (w0, w1)
    sems = (sem0, sem1)
    ssems = (ssem0, ssem1)

    zv = jnp.zeros((16,), jnp.float32)

    def zb_body(r, zc):
        g0[r, pl.ds(0, 16)] = zv
        g0[r, pl.ds(16, 16)] = zv
        return zc
    lax.fori_loop(0, ZR, zb_body, 0, unroll=8)
    for q in range(RPT // ZR):
        z0 = pl.multiple_of(s * RPT + q * ZR, ZR)
        pltpu.sync_copy(g0.at[pl.ds(0, ZR)], acc.at[pl.ds(z0, ZR)])
    plsc.subcore_barrier()

    tbase = s * EPT

    def load_idx(k):
        rb = pl.multiple_of((tbase + k * SCH * CK) // G, GR * SCH)
        pltpu.sync_copy(srcr_hbm.at[pl.ds(rb, GR * SCH)], sidx)
        pltpu.sync_copy(dstr_hbm.at[c, pl.ds(rb, GR * SCH)], didx)

    def wp_slice(k, p):
        ci = s * NCHUNK + k * SCH + p
        b = lax.shift_right_logical(ci, 3)
        cb = lax.bitwise_and(ci, 7)
        q = lax.shift_right_logical(cb, 2)
        pstart = pl.multiple_of(b * EB2 + lax.bitwise_and(cb, 3) * CK, CK)
        return wp_hbm.at[pl.ds(pstart, CK), pl.ds(q * 64 + c * HD, HD)]

    def drain_scatter(par):
        for j in range(GR):
            pltpu.make_async_copy(gb[par].at[pl.ds(j * G, G)],
                                  acc.at[sidx.at[j]], ssems[par]).wait()

    def issue(k, p, par, drain):
        if drain:
            drain_scatter(par)
        for j in range(GR):
            pltpu.async_copy(xi2_hbm.at[didx.at[GR * p + j]],
                             gb[par].at[pl.ds(j * G, G)], sems[par])
        pltpu.async_copy(wp_slice(k, p), wb[par], sems[par])

    def wait(k, p, par):
        for j in range(GR):
            pltpu.make_async_copy(xi2_hbm.at[didx.at[GR * p + j]],
                                  gb[par].at[pl.ds(j * G, G)], sems[par]).wait()
        pltpu.make_async_copy(wp_slice(k, p), wb[par], sems[par]).wait()

    load_idx(0)
    issue(0, 0, 0, False)

    def sup(k, carry):
        for p in range(SCH):
            par = p & 1
            if p < SCH - 1:
                issue(k, p + 1, 1 - par, p != 0)
            wait(k, p, par)
            g, w = gb[par], wb[par]

            def mul(r0, mc):
                for u in range(8):
                    r = r0 * 8 + u
                    g[r, pl.ds(0, 16)] = g[r, pl.ds(0, 16)] * w[r, pl.ds(0, 16)]
                    g[r, pl.ds(16, 16)] = g[r, pl.ds(16, 16)] * w[r, pl.ds(16, 16)]
                return mc
            lax.fori_loop(0, CK // 8, mul, 0)

            for j in range(GR):
                pltpu.async_copy(g.at[pl.ds(j * G, G)],
                                 acc.at[sidx.at[GR * p + j]], ssems[par],
                                 add=True)
            if p == SCH - 1:
                @pl.when(k + 1 < NSUP)
                def _next():
                    drain_scatter(par)
                    load_idx(k + 1)
                    issue(k + 1, 0, 1 - par, True)
        return carry

    lax.fori_loop(0, NSUP, sup, 0)
    drain_scatter(0)
    drain_scatter(1)
    plsc.subcore_barrier()
    for q in range(RPT // ZR):
        r0 = pl.multiple_of(s * RPT + q * ZR, ZR)
        pltpu.sync_copy(acc.at[pl.ds(r0, ZR)], out_hbm.at[pl.ds(r0, ZR), c])


def _sc_conv(xi_h, wp, srcr, dstr):
    mesh = plsc.VectorSubcoreMesh(core_axis_name="c", subcore_axis_name="s")
    return pl.kernel(
        _sc_conv_body,
        out_type=jax.ShapeDtypeStruct((NPAD, NC, HD), jnp.float32),
        mesh=mesh,
        compiler_params=pltpu.CompilerParams(use_tc_tiling_on_sc=False),
        scratch_types=[
            pltpu.VMEM((GR * SCH, G), jnp.int32),
            pltpu.VMEM((GR * SCH, G), jnp.int32),
            pltpu.VMEM((CK, HD), jnp.float32),
            pltpu.VMEM((CK, HD), jnp.float32),
            pltpu.VMEM((CK, HD), jnp.float32),
            pltpu.VMEM((CK, HD), jnp.float32),
            pltpu.VMEM_SHARED((NPAD, HD), jnp.float32),
            pltpu.SemaphoreType.DMA,
            pltpu.SemaphoreType.DMA,
            pltpu.SemaphoreType.DMA,
            pltpu.SemaphoreType.DMA,
        ],
    )(xi_h.reshape(2 * NP2, HD), wp, srcr, dstr)


def kernel(species, edge_src, edge_dst, distances, switch,
           W_sp, aw1_W, aw1_b, f0_W, f0_b, f1_W, f1_b, f2_W, f2_b,
           aw2_W, aw2_b, aw3_W, aw3_b):
    species = species.astype(jnp.int32)
    edge_src = edge_src.astype(jnp.int32)
    edge_dst = edge_dst.astype(jnp.int32)

    wsp_pad = jnp.zeros((128, DIM), jnp.float32).at[:NSPEC].set(W_sp)
    sp2 = jnp.pad(species, (0, NP2 - N)).reshape(NP2 // 2, 2)
    xi2p = _embed(sp2, wsp_pad)

    d3 = jnp.pad(distances, (0, EPAD - E)).reshape(EPAD // EB, 1, EB)
    s3 = jnp.pad(switch, (0, EPAD - E)).reshape(EPAD // EB, 1, EB)
    (wp0,) = _filter([0], d3, s3, f0_W, f0_b, f1_W, f1_b, f2_W, f2_b)

    srcr = jnp.pad(edge_src, (0, EPAD - E)).reshape(EPAD // G, G)
    dstr = jnp.pad(jnp.stack([2 * edge_dst, 2 * edge_dst + 1]),
                   ((0, 0), (0, EPAD - E))).reshape(NC, EPAD // G, G)

    y2p = _node_in(xi2p, aw1_W[0], aw1_b[0])
    acc = _sc_conv(y2p, wp0, srcr, dstr)
    wp1, wp2 = _filter([1, 2], d3, s3, f0_W, f0_b, f1_W, f1_b, f2_W, f2_b)
    xi2p = _node_out(acc.reshape(NP2 // 2, 128), xi2p,
                     aw2_W[0], aw2_b[0], aw3_W[0], aw3_b[0])
    for l, wpl in ((1, wp1), (2, wp2)):
        y2p = _node_in(xi2p, aw1_W[l], aw1_b[l])
        acc = _sc_conv(y2p, wpl, srcr, dstr)
        xi2p = _node_out(acc.reshape(NP2 // 2, 128), xi2p,
                         aw2_W[l], aw2_b[l], aw3_W[l], aw3_b[l])
    return xi2p.reshape(NP2, DIM)[:N]
